# baseline (speedup 1.0000x reference)
import functools

import jax
import jax.numpy as jnp
from jax import lax
from jax.experimental import pallas as pl
from jax.experimental.pallas import tpu as pltpu
from jax.experimental.pallas import tpu_sc as plsc

NPOL = 8
DM = 1024
DF = 4096
AD = 32
APAD = 128
NT = 8192

BT = 1024
NB = NT // BT + NPOL
PADN = NB * BT
FB = 1024
NFB = DF // FB
TRASH = NT

NW = 32
RPW = PADN // NW
CH = 32
NCH = RPW // CH

@functools.cache
def _build_gather_sc():
    mesh = plsc.VectorSubcoreMesh(core_axis_name="c", subcore_axis_name="s")

    @functools.partial(
        pl.kernel,
        mesh=mesh,
        out_type=(
            jax.ShapeDtypeStruct((PADN, DM), jnp.float32),
            jax.ShapeDtypeStruct((PADN, APAD), jnp.float32),
        ),
        scratch_types=[
            pltpu.VMEM((RPW,), jnp.int32),
            pltpu.VMEM((CH, DM), jnp.float32),
            pltpu.VMEM((CH, DM), jnp.float32),
            pltpu.VMEM((CH, APAD), jnp.float32),
            pltpu.VMEM((CH, APAD), jnp.float32),
            pltpu.SemaphoreType.DMA,
            pltpu.SemaphoreType.DMA,
        ],
    )
    def gather_k(z_hbm, a_hbm, src_hbm, oz_hbm, oa_hbm,
                 idx_v, zb0, zb1, ab0, ab1, semg, sems):
        wid = lax.axis_index("s") * 2 + lax.axis_index("c")
        base = wid * RPW
        pltpu.sync_copy(src_hbm.at[pl.ds(base, RPW)], idx_v)
        zb, ab = (zb0, zb1), (ab0, ab1)
        gath = [None, None]
        stor = [None, None]
        for c in range(NCH + 1):
            b = c % 2
            if c < NCH:
                if stor[b] is not None:
                    stor[b][0].wait()
                    stor[b][1].wait()
                gath[b] = (
                    pltpu.async_copy(
                        z_hbm.at[idx_v.at[pl.ds(c * CH, CH)]], zb[b], semg),
                    pltpu.async_copy(
                        a_hbm.at[idx_v.at[pl.ds(c * CH, CH)]], ab[b], semg),
                )
            if c >= 1:
                pb = (c - 1) % 2
                off = base + (c - 1) * CH
                gath[pb][0].wait()
                gath[pb][1].wait()
                stor[pb] = (
                    pltpu.async_copy(zb[pb], oz_hbm.at[pl.ds(off, CH)], sems),
                    pltpu.async_copy(ab[pb], oa_hbm.at[pl.ds(off, CH)], sems),
                )
        for s in stor:
            if s is not None:
                s[0].wait()
                s[1].wait()

    return gather_k


@functools.cache
def _build_scatter_sc():
    mesh = plsc.VectorSubcoreMesh(core_axis_name="c", subcore_axis_name="s")

    @functools.partial(
        pl.kernel,
        mesh=mesh,
        out_type=jax.ShapeDtypeStruct((NT + 8, DM), jnp.float32),
        scratch_types=[
            pltpu.VMEM((NCH, CH), jnp.int32),
            pltpu.VMEM((CH, DM), jnp.float32),
            pltpu.VMEM((CH, DM), jnp.float32),
            pltpu.SemaphoreType.DMA,
            pltpu.SemaphoreType.DMA,
        ],
    )
    def scatter_k(ys_hbm, dst3_hbm, out_hbm, idx_v, rb0, rb1, seml, semsc):
        wid = lax.axis_index("s") * 2 + lax.axis_index("c")
        base = wid * RPW
        pltpu.sync_copy(dst3_hbm.at[wid], idx_v)
        rb = (rb0, rb1)
        load = [None, None]
        scat = [None, None]
        for c in range(NCH + 1):
            b = c % 2
            if c < NCH:
                if scat[b] is not None:
                    scat[b].wait()
                load[b] = pltpu.async_copy(
                    ys_hbm.at[pl.ds(base + c * CH, CH)], rb[b], seml)
            if c >= 1:
                pb = (c - 1) % 2
                load[pb].wait()
                scat[pb] = pltpu.async_copy(
                    rb[pb], out_hbm.at[idx_v.at[c - 1]], semsc)
        for s in scat:
            if s is not None:
                s.wait()

    return scatter_k


def _gather_sc(latents, actions, src):
    return _build_gather_sc()(latents, actions, src)


def _scatter_sc(ys, dst):
    return _build_scatter_sc()(ys, dst.reshape(NW, NCH, CH))


def _mlp_body(be_ref, xz_ref, xa_ref, w1z_ref, w1a_ref, b1_ref, w2_ref,
              b2_ref, o_ref):
    j = pl.program_id(1)
    bf = jnp.bfloat16
    h = lax.dot_general(xz_ref[...].astype(bf), w1z_ref[0].astype(bf),
                        (((1,), (0,)), ((), ())),
                        preferred_element_type=jnp.float32)
    w1a = jnp.concatenate(
        [w1a_ref[0], jnp.zeros((APAD - AD, FB), jnp.float32)], axis=0)
    h = h + lax.dot_general(xa_ref[...].astype(bf), w1a.astype(bf),
                            (((1,), (0,)), ((), ())),
                            preferred_element_type=jnp.float32)
    h = jnp.maximum(h + b1_ref[0], 0.0)
    y = lax.dot_general(h.astype(bf), w2_ref[0].astype(bf),
                        (((1,), (0,)), ((), ())),
                        preferred_element_type=jnp.float32)

    @pl.when(j == 0)
    def _():
        o_ref[...] = y + b2_ref[0]

    @pl.when(j != 0)
    def _():
        o_ref[...] = o_ref[...] + y


def _mlp_grid_spec():
    return pltpu.PrefetchScalarGridSpec(
        num_scalar_prefetch=1,
        grid=(NB, NFB),
        in_specs=[
            pl.BlockSpec((BT, DM), lambda i, j, be: (i, 0)),
            pl.BlockSpec((BT, APAD), lambda i, j, be: (i, 0)),
            pl.BlockSpec((1, DM, FB), lambda i, j, be: (be[i], 0, j)),
            pl.BlockSpec((1, AD, FB), lambda i, j, be: (be[i], DM // AD, j)),
            pl.BlockSpec((1, 1, FB), lambda i, j, be: (be[i], 0, j)),
            pl.BlockSpec((1, FB, DM), lambda i, j, be: (be[i], j, 0)),
            pl.BlockSpec((1, 1, DM), lambda i, j, be: (be[i], 0, 0)),
        ],
        out_specs=pl.BlockSpec((BT, DM), lambda i, j, be: (i, 0)),
    )


def _grouped_mlp(be, xz, xa, W1, b1, W2, b2):
    return pl.pallas_call(
        _mlp_body,
        grid_spec=_mlp_grid_spec(),
        out_shape=jax.ShapeDtypeStruct((PADN, DM), jnp.float32),
        compiler_params=pltpu.CompilerParams(
            dimension_semantics=("arbitrary", "arbitrary")),
    )(be, xz, xa, W1, W1, b1.reshape(NPOL, 1, DF), W2, b2.reshape(NPOL, 1, DM))


def _route(pid):
    pid = pid.astype(jnp.int32)
    order = jnp.argsort(pid).astype(jnp.int32)
    counts = jnp.bincount(pid, length=NPOL).astype(jnp.int32)
    offs = jnp.concatenate(
        [jnp.zeros((1,), jnp.int32), jnp.cumsum(counts)[:-1].astype(jnp.int32)])
    nbe = (counts + BT - 1) // BT
    cnb = jnp.cumsum(nbe).astype(jnp.int32)
    bid = jnp.arange(NB, dtype=jnp.int32)
    be = jnp.searchsorted(cnb, bid, side="right").astype(jnp.int32)
    be = jnp.minimum(be, NPOL - 1)
    bstart = jnp.concatenate(
        [jnp.zeros((1,), jnp.int32), cnb[:-1]])
    brank = bid - bstart[be]
    p = jnp.arange(PADN, dtype=jnp.int32)
    blk = p // BT
    e = be[blk]
    r = brank[blk] * BT + (p % BT)
    valid = r < counts[e]
    src = jnp.where(valid, order[jnp.clip(offs[e] + r, 0, NT - 1)], 0)
    dst = jnp.where(valid, src, TRASH).astype(jnp.int32)
    return be, src.astype(jnp.int32), dst


def kernel(latents, policy_indices, actions, W1, b1, W2, b2):
    be, src, dst = _route(policy_indices)
    ap = jnp.pad(actions, ((0, 0), (0, APAD - AD)))
    xz, xa = _gather_sc(latents, ap, src)
    ys = _grouped_mlp(be, xz, xa, W1, b1, W2, b2)
    out = _scatter_sc(ys, dst)
    return out[:NT]


# speedup vs baseline: 2.1193x; 2.1193x over previous
import functools

import jax
import jax.numpy as jnp
from jax import lax
from jax.experimental import pallas as pl
from jax.experimental.pallas import tpu as pltpu
from jax.experimental.pallas import tpu_sc as plsc

NPOL = 8
DM = 1024
DF = 4096
AD = 32
APAD = 128
NT = 8192

BT = 1024
NB = NT // BT + NPOL
PADN = NB * BT
FB = 1024
NFB = DF // FB

NW = 32
RPW = PADN // NW
CH = 32
NCH = RPW // CH

@functools.cache
def _build_gather_sc():
    mesh = plsc.VectorSubcoreMesh(core_axis_name="c", subcore_axis_name="s")

    @functools.partial(
        pl.kernel,
        mesh=mesh,
        out_type=(
            jax.ShapeDtypeStruct((PADN, DM), jnp.float32),
            jax.ShapeDtypeStruct((PADN, APAD), jnp.float32),
        ),
        scratch_types=[
            pltpu.VMEM((RPW,), jnp.int32),
            pltpu.VMEM((CH, DM), jnp.float32),
            pltpu.VMEM((CH, DM), jnp.float32),
            pltpu.VMEM((CH, APAD), jnp.float32),
            pltpu.VMEM((CH, APAD), jnp.float32),
            pltpu.SemaphoreType.DMA,
            pltpu.SemaphoreType.DMA,
        ],
    )
    def gather_k(z_hbm, a_hbm, src_hbm, oz_hbm, oa_hbm,
                 idx_v, zb0, zb1, ab0, ab1, semg, sems):
        wid = lax.axis_index("s") * 2 + lax.axis_index("c")
        base = wid * RPW
        pltpu.sync_copy(src_hbm.at[pl.ds(base, RPW)], idx_v)
        zb, ab = (zb0, zb1), (ab0, ab1)
        gath = [None, None]
        stor = [None, None]
        for c in range(NCH + 1):
            b = c % 2
            if c < NCH:
                if stor[b] is not None:
                    stor[b][0].wait()
                    stor[b][1].wait()
                gath[b] = (
                    pltpu.async_copy(
                        z_hbm.at[idx_v.at[pl.ds(c * CH, CH)]], zb[b], semg),
                    pltpu.async_copy(
                        a_hbm.at[idx_v.at[pl.ds(c * CH, CH)]], ab[b], semg),
                )
            if c >= 1:
                pb = (c - 1) % 2
                off = base + (c - 1) * CH
                gath[pb][0].wait()
                gath[pb][1].wait()
                stor[pb] = (
                    pltpu.async_copy(zb[pb], oz_hbm.at[pl.ds(off, CH)], sems),
                    pltpu.async_copy(ab[pb], oa_hbm.at[pl.ds(off, CH)], sems),
                )
        for s in stor:
            if s is not None:
                s[0].wait()
                s[1].wait()

    return gather_k


@functools.cache
def _build_scatter_sc():
    mesh = plsc.VectorSubcoreMesh(core_axis_name="c", subcore_axis_name="s")

    @functools.partial(
        pl.kernel,
        mesh=mesh,
        out_type=jax.ShapeDtypeStruct((PADN, DM), jnp.float32),
        scratch_types=[
            pltpu.VMEM((NCH, CH), jnp.int32),
            pltpu.VMEM((CH, DM), jnp.float32),
            pltpu.VMEM((CH, DM), jnp.float32),
            pltpu.SemaphoreType.DMA,
            pltpu.SemaphoreType.DMA,
        ],
    )
    def scatter_k(ys_hbm, dst3_hbm, out_hbm, idx_v, rb0, rb1, seml, semsc):
        wid = lax.axis_index("s") * 2 + lax.axis_index("c")
        base = wid * RPW
        pltpu.sync_copy(dst3_hbm.at[wid], idx_v)
        rb = (rb0, rb1)
        load = [None, None]
        scat = [None, None]
        for c in range(NCH + 1):
            b = c % 2
            if c < NCH:
                if scat[b] is not None:
                    scat[b].wait()
                load[b] = pltpu.async_copy(
                    ys_hbm.at[pl.ds(base + c * CH, CH)], rb[b], seml)
            if c >= 1:
                pb = (c - 1) % 2
                load[pb].wait()
                scat[pb] = pltpu.async_copy(
                    rb[pb], out_hbm.at[idx_v.at[c - 1]], semsc)
        for s in scat:
            if s is not None:
                s.wait()

    return scatter_k


def _gather_sc(latents, actions, src):
    return _build_gather_sc()(latents, actions, src)


def _scatter_sc(ys, dst):
    return _build_scatter_sc()(ys, dst.reshape(NW, NCH, CH))


def _mlp_body(be_ref, xz_ref, xa_ref, w1z_ref, w1a_ref, b1_ref, w2_ref,
              b2_ref, o_ref):
    j = pl.program_id(1)
    bf = jnp.bfloat16
    h = lax.dot_general(xz_ref[...].astype(bf), w1z_ref[0].astype(bf),
                        (((1,), (0,)), ((), ())),
                        preferred_element_type=jnp.float32)
    w1a = jnp.concatenate(
        [w1a_ref[0], jnp.zeros((APAD - AD, FB), jnp.float32)], axis=0)
    h = h + lax.dot_general(xa_ref[...].astype(bf), w1a.astype(bf),
                            (((1,), (0,)), ((), ())),
                            preferred_element_type=jnp.float32)
    h = jnp.maximum(h + b1_ref[0], 0.0)
    y = lax.dot_general(h.astype(bf), w2_ref[0].astype(bf),
                        (((1,), (0,)), ((), ())),
                        preferred_element_type=jnp.float32)

    @pl.when(j == 0)
    def _():
        o_ref[...] = y + b2_ref[0]

    @pl.when(j != 0)
    def _():
        o_ref[...] = o_ref[...] + y


def _mlp_grid_spec():
    return pltpu.PrefetchScalarGridSpec(
        num_scalar_prefetch=1,
        grid=(NB, NFB),
        in_specs=[
            pl.BlockSpec((BT, DM), lambda i, j, be: (i, 0)),
            pl.BlockSpec((BT, APAD), lambda i, j, be: (i, 0)),
            pl.BlockSpec((1, DM, FB), lambda i, j, be: (be[i], 0, j)),
            pl.BlockSpec((1, AD, FB), lambda i, j, be: (be[i], DM // AD, j)),
            pl.BlockSpec((1, 1, FB), lambda i, j, be: (be[i], 0, j)),
            pl.BlockSpec((1, FB, DM), lambda i, j, be: (be[i], j, 0)),
            pl.BlockSpec((1, 1, DM), lambda i, j, be: (be[i], 0, 0)),
        ],
        out_specs=pl.BlockSpec((BT, DM), lambda i, j, be: (i, 0)),
    )


def _grouped_mlp(be, xz, xa, W1, b1, W2, b2):
    return pl.pallas_call(
        _mlp_body,
        grid_spec=_mlp_grid_spec(),
        out_shape=jax.ShapeDtypeStruct((PADN, DM), jnp.float32),
        compiler_params=pltpu.CompilerParams(
            dimension_semantics=("arbitrary", "arbitrary")),
    )(be, xz, xa, W1, W1, b1.reshape(NPOL, 1, DF), W2, b2.reshape(NPOL, 1, DM))


def _route(pid):
    pid = pid.astype(jnp.int32)
    order = jnp.argsort(pid).astype(jnp.int32)
    counts = jnp.bincount(pid, length=NPOL).astype(jnp.int32)
    offs = jnp.concatenate(
        [jnp.zeros((1,), jnp.int32), jnp.cumsum(counts)[:-1].astype(jnp.int32)])
    nbe = (counts + BT - 1) // BT
    cnb = jnp.cumsum(nbe).astype(jnp.int32)
    bid = jnp.arange(NB, dtype=jnp.int32)
    be = jnp.searchsorted(cnb, bid, side="right").astype(jnp.int32)
    be = jnp.minimum(be, NPOL - 1)
    bstart = jnp.concatenate(
        [jnp.zeros((1,), jnp.int32), cnb[:-1]])
    brank = bid - bstart[be]
    p = jnp.arange(PADN, dtype=jnp.int32)
    blk = p // BT
    e = be[blk]
    r = brank[blk] * BT + (p % BT)
    valid = r < counts[e]
    src = jnp.where(valid, order[jnp.clip(offs[e] + r, 0, NT - 1)], p % NT)
    dst = jnp.where(valid, src, NT + p % (PADN - NT)).astype(jnp.int32)
    return be, src.astype(jnp.int32), dst


def kernel(latents, policy_indices, actions, W1, b1, W2, b2):
    be, src, dst = _route(policy_indices)
    ap = jnp.pad(actions, ((0, 0), (0, APAD - AD)))
    xz, xa = _gather_sc(latents, ap, src)
    ys = _grouped_mlp(be, xz, xa, W1, b1, W2, b2)
    out = _scatter_sc(ys, dst)
    return out[:NT]


# speedup vs baseline: 3.5310x; 1.6661x over previous
import functools

import jax
import jax.numpy as jnp
from jax import lax
from jax.experimental import pallas as pl
from jax.experimental.pallas import tpu as pltpu
from jax.experimental.pallas import tpu_sc as plsc

NPOL = 8
DM = 1024
DF = 4096
AD = 32
APAD = 128
NT = 8192

BT = 1024
NB = NT // BT + NPOL
PADN = NB * BT
FB = 1024
NFB = DF // FB

NW = 32
RPT = NT // NW
CH = 32
NCH = RPT // CH


@functools.cache
def _build_scatter_in():
    mesh = plsc.VectorSubcoreMesh(core_axis_name="c", subcore_axis_name="s")

    @functools.partial(
        pl.kernel,
        mesh=mesh,
        out_type=(
            jax.ShapeDtypeStruct((PADN, DM), jnp.float32),
            jax.ShapeDtypeStruct((PADN, APAD), jnp.float32),
        ),
        scratch_types=[
            pltpu.VMEM((NCH, CH), jnp.int32),
            pltpu.VMEM((CH, DM), jnp.float32),
            pltpu.VMEM((CH, DM), jnp.float32),
            pltpu.VMEM((CH, APAD), jnp.float32),
            pltpu.VMEM((CH, APAD), jnp.float32),
            pltpu.SemaphoreType.DMA,
            pltpu.SemaphoreType.DMA,
        ],
    )
    def scatter_k(z_hbm, a_hbm, pos3_hbm, xz_hbm, xa_hbm,
                  idx_v, zb0, zb1, ab0, ab1, seml, semsc):
        wid = lax.axis_index("s") * 2 + lax.axis_index("c")
        base = wid * RPT
        pltpu.sync_copy(pos3_hbm.at[wid], idx_v)
        zb, ab = (zb0, zb1), (ab0, ab1)
        load = [None, None]
        scat = [None, None]
        for c in range(NCH + 1):
            b = c % 2
            if c < NCH:
                if scat[b] is not None:
                    scat[b][0].wait()
                    scat[b][1].wait()
                off = base + c * CH
                load[b] = (
                    pltpu.async_copy(z_hbm.at[pl.ds(off, CH)], zb[b], seml),
                    pltpu.async_copy(a_hbm.at[pl.ds(off, CH)], ab[b], seml),
                )
            if c >= 1:
                pb = (c - 1) % 2
                load[pb][0].wait()
                load[pb][1].wait()
                scat[pb] = (
                    pltpu.async_copy(zb[pb], xz_hbm.at[idx_v.at[c - 1]],
                                     semsc),
                    pltpu.async_copy(ab[pb], xa_hbm.at[idx_v.at[c - 1]],
                                     semsc),
                )
        for s in scat:
            if s is not None:
                s[0].wait()
                s[1].wait()

    return scatter_k


@functools.cache
def _build_gather_out():
    mesh = plsc.VectorSubcoreMesh(core_axis_name="c", subcore_axis_name="s")

    @functools.partial(
        pl.kernel,
        mesh=mesh,
        out_type=jax.ShapeDtypeStruct((NT, DM), jnp.float32),
        scratch_types=[
            pltpu.VMEM((RPT,), jnp.int32),
            pltpu.VMEM((CH, DM), jnp.float32),
            pltpu.VMEM((CH, DM), jnp.float32),
            pltpu.SemaphoreType.DMA,
            pltpu.SemaphoreType.DMA,
        ],
    )
    def gather_k(ys_hbm, pos_hbm, out_hbm, idx_v, rb0, rb1, semg, sems):
        wid = lax.axis_index("s") * 2 + lax.axis_index("c")
        base = wid * RPT
        pltpu.sync_copy(pos_hbm.at[pl.ds(base, RPT)], idx_v)
        rb = (rb0, rb1)
        gath = [None, None]
        stor = [None, None]
        for c in range(NCH + 1):
            b = c % 2
            if c < NCH:
                if stor[b] is not None:
                    stor[b].wait()
                gath[b] = pltpu.async_copy(
                    ys_hbm.at[idx_v.at[pl.ds(c * CH, CH)]], rb[b], semg)
            if c >= 1:
                pb = (c - 1) % 2
                gath[pb].wait()
                stor[pb] = pltpu.async_copy(
                    rb[pb], out_hbm.at[pl.ds(base + (c - 1) * CH, CH)], sems)
        for s in stor:
            if s is not None:
                s.wait()

    return gather_k


def _scatter_in(latents, actions_padded, pos):
    return _build_scatter_in()(latents, actions_padded,
                               pos.reshape(NW, NCH, CH))


def _gather_out(ys, pos):
    return _build_gather_out()(ys, pos)


def _mlp_body(be_ref, ie_ref, xz_ref, xa_ref, w1z_ref, w1a_ref, b1_ref,
              w2_ref, b2_ref, o_ref):
    i = pl.program_id(0)
    j = pl.program_id(1)

    @pl.when(ie_ref[i] == i)
    def _():
        bf = jnp.bfloat16
        h = lax.dot_general(xz_ref[...].astype(bf), w1z_ref[0].astype(bf),
                            (((1,), (0,)), ((), ())),
                            preferred_element_type=jnp.float32)
        w1a = jnp.concatenate(
            [w1a_ref[0], jnp.zeros((APAD - AD, FB), jnp.float32)], axis=0)
        h = h + lax.dot_general(xa_ref[...].astype(bf), w1a.astype(bf),
                                (((1,), (0,)), ((), ())),
                                preferred_element_type=jnp.float32)
        h = jnp.maximum(h + b1_ref[0], 0.0)
        y = lax.dot_general(h.astype(bf), w2_ref[0].astype(bf),
                            (((1,), (0,)), ((), ())),
                            preferred_element_type=jnp.float32)

        @pl.when(j == 0)
        def _():
            o_ref[...] = y + b2_ref[0]

        @pl.when(j != 0)
        def _():
            o_ref[...] = o_ref[...] + y


def _mlp_grid_spec():
    return pltpu.PrefetchScalarGridSpec(
        num_scalar_prefetch=2,
        grid=(NB, NFB),
        in_specs=[
            pl.BlockSpec((BT, DM), lambda i, j, be, ie: (ie[i], 0)),
            pl.BlockSpec((BT, APAD), lambda i, j, be, ie: (ie[i], 0)),
            pl.BlockSpec((1, DM, FB), lambda i, j, be, ie: (be[i], 0, j)),
            pl.BlockSpec((1, AD, FB), lambda i, j, be, ie: (be[i], DM // AD, j)),
            pl.BlockSpec((1, 1, FB), lambda i, j, be, ie: (be[i], 0, j)),
            pl.BlockSpec((1, FB, DM), lambda i, j, be, ie: (be[i], j, 0)),
            pl.BlockSpec((1, 1, DM), lambda i, j, be, ie: (be[i], 0, 0)),
        ],
        out_specs=pl.BlockSpec((BT, DM), lambda i, j, be, ie: (i, 0)),
    )


def _grouped_mlp(be, ieff, xz, xa, W1, b1, W2, b2):
    return pl.pallas_call(
        _mlp_body,
        grid_spec=_mlp_grid_spec(),
        out_shape=jax.ShapeDtypeStruct((PADN, DM), jnp.float32),
        compiler_params=pltpu.CompilerParams(
            dimension_semantics=("arbitrary", "arbitrary")),
    )(be, ieff, xz, xa, W1, W1, b1.reshape(NPOL, 1, DF), W2,
      b2.reshape(NPOL, 1, DM))


def _route(pid):
    pid = pid.astype(jnp.int32)
    onehot = (pid[:, None] == jnp.arange(NPOL, dtype=jnp.int32)[None, :])
    cum = jnp.cumsum(onehot.astype(jnp.int32), axis=0)
    rank = jnp.take_along_axis(cum, pid[:, None], axis=1)[:, 0] - 1
    counts = cum[-1]
    nbe = (counts + BT - 1) // BT
    cnb = jnp.cumsum(nbe).astype(jnp.int32)
    total = cnb[-1]
    bstart = (cnb - nbe).astype(jnp.int32)
    pos = (bstart[pid] * BT + rank).astype(jnp.int32)
    bid = jnp.arange(NB, dtype=jnp.int32)
    ieff = jnp.minimum(bid, total - 1)
    be = jnp.searchsorted(cnb, ieff, side="right").astype(jnp.int32)
    return be, ieff.astype(jnp.int32), pos


def kernel(latents, policy_indices, actions, W1, b1, W2, b2):
    be, ieff, pos = _route(policy_indices)
    ap = jnp.pad(actions, ((0, 0), (0, APAD - AD)))
    xz, xa = _scatter_in(latents, ap, pos)
    ys = _grouped_mlp(be, ieff, xz, xa, W1, b1, W2, b2)
    return _gather_out(ys, pos)


# speedup vs baseline: 3.8648x; 1.0945x over previous
import functools

import jax
import jax.numpy as jnp
from jax import lax
from jax.experimental import pallas as pl
from jax.experimental.pallas import tpu as pltpu
from jax.experimental.pallas import tpu_sc as plsc

NPOL = 8
DM = 1024
DF = 4096
AD = 32
APAD = 128
NT = 8192

BT = 1024
NB = NT // BT + NPOL
PADN = NB * BT
FB = 1024
NFB = DF // FB

NW = 32
RPT = NT // NW
CH = 32
NCH = RPT // CH


@functools.cache
def _build_scatter_in():
    mesh = plsc.VectorSubcoreMesh(core_axis_name="c", subcore_axis_name="s")

    @functools.partial(
        pl.kernel,
        mesh=mesh,
        out_type=(
            jax.ShapeDtypeStruct((PADN, DM), jnp.float32),
            jax.ShapeDtypeStruct((PADN, APAD), jnp.float32),
        ),
        scratch_types=[
            pltpu.VMEM((NCH, CH), jnp.int32),
            pltpu.VMEM((CH, DM), jnp.float32),
            pltpu.VMEM((CH, DM), jnp.float32),
            pltpu.VMEM((CH, APAD), jnp.float32),
            pltpu.VMEM((CH, APAD), jnp.float32),
            pltpu.SemaphoreType.DMA,
            pltpu.SemaphoreType.DMA,
        ],
    )
    def scatter_k(z_hbm, a_hbm, pos3_hbm, xz_hbm, xa_hbm,
                  idx_v, zb0, zb1, ab0, ab1, seml, semsc):
        wid = lax.axis_index("s") * 2 + lax.axis_index("c")
        base = wid * RPT
        pltpu.sync_copy(pos3_hbm.at[wid], idx_v)
        zb, ab = (zb0, zb1), (ab0, ab1)
        load = [None, None]
        scat = [None, None]
        for c in range(NCH + 1):
            b = c % 2
            if c < NCH:
                if scat[b] is not None:
                    scat[b][0].wait()
                    scat[b][1].wait()
                off = base + c * CH
                load[b] = (
                    pltpu.async_copy(z_hbm.at[pl.ds(off, CH)], zb[b], seml),
                    pltpu.async_copy(a_hbm.at[pl.ds(off, CH)], ab[b], seml),
                )
            if c >= 1:
                pb = (c - 1) % 2
                load[pb][0].wait()
                load[pb][1].wait()
                scat[pb] = (
                    pltpu.async_copy(zb[pb], xz_hbm.at[idx_v.at[c - 1]],
                                     semsc),
                    pltpu.async_copy(ab[pb], xa_hbm.at[idx_v.at[c - 1]],
                                     semsc),
                )
        for s in scat:
            if s is not None:
                s[0].wait()
                s[1].wait()

    return scatter_k


@functools.cache
def _build_gather_out():
    mesh = plsc.VectorSubcoreMesh(core_axis_name="c", subcore_axis_name="s")

    @functools.partial(
        pl.kernel,
        mesh=mesh,
        out_type=jax.ShapeDtypeStruct((NT, DM), jnp.float32),
        scratch_types=[
            pltpu.VMEM((RPT,), jnp.int32),
            pltpu.VMEM((CH, DM), jnp.float32),
            pltpu.VMEM((CH, DM), jnp.float32),
            pltpu.SemaphoreType.DMA,
            pltpu.SemaphoreType.DMA,
        ],
    )
    def gather_k(ys_hbm, pos_hbm, out_hbm, idx_v, rb0, rb1, semg, sems):
        wid = lax.axis_index("s") * 2 + lax.axis_index("c")
        base = wid * RPT
        pltpu.sync_copy(pos_hbm.at[pl.ds(base, RPT)], idx_v)
        rb = (rb0, rb1)
        gath = [None, None]
        stor = [None, None]
        for c in range(NCH + 1):
            b = c % 2
            if c < NCH:
                if stor[b] is not None:
                    stor[b].wait()
                gath[b] = pltpu.async_copy(
                    ys_hbm.at[idx_v.at[pl.ds(c * CH, CH)]], rb[b], semg)
            if c >= 1:
                pb = (c - 1) % 2
                gath[pb].wait()
                stor[pb] = pltpu.async_copy(
                    rb[pb], out_hbm.at[pl.ds(base + (c - 1) * CH, CH)], sems)
        for s in stor:
            if s is not None:
                s.wait()

    return gather_k


def _scatter_in(latents, actions_padded, pos):
    return _build_scatter_in()(latents, actions_padded,
                               pos.reshape(NW, NCH, CH))


def _gather_out(ys, pos):
    return _build_gather_out()(ys, pos)


def _mlp_body(be_ref, ie_ref, js_ref, xz_ref, xa_ref, w1z_ref, w1a_ref,
              b1_ref, w2_ref, b2_ref, o_ref):
    i = pl.program_id(0)
    j = pl.program_id(1)

    @pl.when(ie_ref[i] == i)
    def _():
        bf = jnp.bfloat16
        h = lax.dot_general(xz_ref[...].astype(bf), w1z_ref[0].astype(bf),
                            (((1,), (0,)), ((), ())),
                            preferred_element_type=jnp.float32)
        w1a = jnp.concatenate(
            [w1a_ref[0], jnp.zeros((APAD - AD, FB), jnp.float32)], axis=0)
        h = h + lax.dot_general(xa_ref[...].astype(bf), w1a.astype(bf),
                                (((1,), (0,)), ((), ())),
                                preferred_element_type=jnp.float32)
        h = jnp.maximum(h + b1_ref[0], 0.0)
        y = lax.dot_general(h.astype(bf), w2_ref[0].astype(bf),
                            (((1,), (0,)), ((), ())),
                            preferred_element_type=jnp.float32)

        @pl.when(j == 0)
        def _():
            o_ref[...] = y + b2_ref[0]

        @pl.when(j != 0)
        def _():
            o_ref[...] = o_ref[...] + y


def _mlp_grid_spec():
    return pltpu.PrefetchScalarGridSpec(
        num_scalar_prefetch=3,
        grid=(NB, NFB),
        in_specs=[
            pl.BlockSpec((BT, DM), lambda i, j, be, ie, js: (ie[i], 0)),
            pl.BlockSpec((BT, APAD), lambda i, j, be, ie, js: (ie[i], 0)),
            pl.BlockSpec((1, DM, FB),
                         lambda i, j, be, ie, js: (be[i], 0, js[i, j])),
            pl.BlockSpec((1, AD, FB),
                         lambda i, j, be, ie, js: (be[i], DM // AD, js[i, j])),
            pl.BlockSpec((1, 1, FB),
                         lambda i, j, be, ie, js: (be[i], 0, js[i, j])),
            pl.BlockSpec((1, FB, DM),
                         lambda i, j, be, ie, js: (be[i], js[i, j], 0)),
            pl.BlockSpec((1, 1, DM), lambda i, j, be, ie, js: (be[i], 0, 0)),
        ],
        out_specs=pl.BlockSpec((BT, DM), lambda i, j, be, ie, js: (i, 0)),
    )


def _grouped_mlp(be, ieff, jsel, xz, xa, W1, b1, W2, b2):
    return pl.pallas_call(
        _mlp_body,
        grid_spec=_mlp_grid_spec(),
        out_shape=jax.ShapeDtypeStruct((PADN, DM), jnp.float32),
        compiler_params=pltpu.CompilerParams(
            dimension_semantics=("arbitrary", "arbitrary")),
    )(be, ieff, jsel, xz, xa, W1, W1, b1.reshape(NPOL, 1, DF), W2,
      b2.reshape(NPOL, 1, DM))


def _route(pid):
    pid = pid.astype(jnp.int32)
    onehot = (pid[:, None] == jnp.arange(NPOL, dtype=jnp.int32)[None, :])
    cum = jnp.cumsum(onehot.astype(jnp.int32), axis=0)
    rank = jnp.take_along_axis(cum, pid[:, None], axis=1)[:, 0] - 1
    counts = cum[-1]
    nbe = (counts + BT - 1) // BT
    cnb = jnp.cumsum(nbe).astype(jnp.int32)
    total = cnb[-1]
    bstart = (cnb - nbe).astype(jnp.int32)
    pos = (bstart[pid] * BT + rank).astype(jnp.int32)
    bid = jnp.arange(NB, dtype=jnp.int32)
    ieff = jnp.minimum(bid, total - 1)
    be = jnp.searchsorted(cnb, ieff, side="right").astype(jnp.int32)
    jj = jnp.arange(NFB, dtype=jnp.int32)
    serp = jnp.where((bid[:, None] % 2) == 1, NFB - 1 - jj[None, :],
                     jj[None, :])
    last_je = serp[total - 1, NFB - 1]
    jsel = jnp.where(bid[:, None] < total, serp, last_je).astype(jnp.int32)
    return be, ieff.astype(jnp.int32), jsel, pos


def kernel(latents, policy_indices, actions, W1, b1, W2, b2):
    be, ieff, jsel, pos = _route(policy_indices)
    ap = jnp.pad(actions, ((0, 0), (0, APAD - AD)))
    xz, xa = _scatter_in(latents, ap, pos)
    ys = _grouped_mlp(be, ieff, jsel, xz, xa, W1, b1, W2, b2)
    return _gather_out(ys, pos)
